# Initial kernel scaffold; baseline (speedup 1.0000x reference)
import jax
import jax.numpy as jnp
from jax.experimental import pallas as pl


def kernel(inputs, table):
    raise NotImplementedError("write your pallas kernel here")


# speedup vs baseline: 1.0104x; 1.0104x over previous
import functools

import jax
import jax.numpy as jnp
from jax import lax
from jax.experimental import pallas as pl
from jax.experimental.pallas import tpu as pltpu
from jax.experimental.pallas import tpu_sc as plsc

_EMBED_DIM = 50

_info = plsc.get_sparse_core_info()
_NC = _info.num_cores
_NS = _info.num_subcores
_NW = _NC * _NS

_CHUNK = 128


def _make_gather(total_rows: int):
    rows_per_w = total_rows // _NW
    n_chunks = rows_per_w // _CHUNK
    mesh = plsc.VectorSubcoreMesh(core_axis_name="c", subcore_axis_name="s")

    @functools.partial(
        pl.kernel,
        mesh=mesh,
        compiler_params=pltpu.CompilerParams(use_tc_tiling_on_sc=False),
        out_type=jax.ShapeDtypeStruct((total_rows, _EMBED_DIM), jnp.float32),
        scratch_types=[
            pltpu.VMEM((rows_per_w,), jnp.int32),
            pltpu.VMEM((_CHUNK, _EMBED_DIM), jnp.float32),
            pltpu.SemaphoreType.DMA,
        ],
    )
    def gather_kernel(idx_hbm, table_hbm, out_hbm, idx_v, rows_v, gsem):
        wid = lax.axis_index("s") * _NC + lax.axis_index("c")
        base = wid * rows_per_w
        pltpu.sync_copy(idx_hbm.at[pl.ds(base, rows_per_w)], idx_v)

        def body(g, carry):
            pltpu.async_copy(
                table_hbm.at[idx_v.at[pl.ds(g * _CHUNK, _CHUNK)]],
                rows_v,
                gsem,
            ).wait()
            pltpu.sync_copy(
                rows_v, out_hbm.at[pl.ds(base + g * _CHUNK, _CHUNK)]
            )
            return carry

        lax.fori_loop(0, n_chunks, body, 0)

    return gather_kernel


def kernel(inputs, table):
    batch, hist = inputs.shape
    total = batch * hist
    idx = inputs.reshape(total).astype(jnp.int32)
    out = _make_gather(total)(idx, table)
    return out.reshape(batch, hist, _EMBED_DIM)


# speedup vs baseline: 1.4333x; 1.4186x over previous
import functools

import jax
import jax.numpy as jnp
from jax import lax
from jax.experimental import pallas as pl
from jax.experimental.pallas import tpu as pltpu
from jax.experimental.pallas import tpu_sc as plsc

_EMBED_DIM = 50
_ROW = 128

_info = plsc.get_sparse_core_info()
_NC = _info.num_cores
_NS = _info.num_subcores
_NW = _NC * _NS

_CHUNK = 128


def _make_gather(total_rows: int):
    rows_per_w = total_rows // _NW
    n_chunks = rows_per_w // _CHUNK
    mesh = plsc.VectorSubcoreMesh(core_axis_name="c", subcore_axis_name="s")

    @functools.partial(
        pl.kernel,
        mesh=mesh,
        out_type=jax.ShapeDtypeStruct((total_rows, _ROW), jnp.float32),
        scratch_types=[
            pltpu.VMEM((rows_per_w,), jnp.int32),
            pltpu.VMEM((_CHUNK, _ROW), jnp.float32),
            pltpu.SemaphoreType.DMA,
        ],
    )
    def gather_kernel(idx_hbm, table_hbm, out_hbm, idx_v, rows_v, gsem):
        wid = lax.axis_index("s") * _NC + lax.axis_index("c")
        base = wid * rows_per_w
        pltpu.sync_copy(idx_hbm.at[pl.ds(base, rows_per_w)], idx_v)

        def body(g, carry):
            pltpu.async_copy(
                table_hbm.at[idx_v.at[pl.ds(g * _CHUNK, _CHUNK)]],
                rows_v,
                gsem,
            ).wait()
            pltpu.sync_copy(
                rows_v,
                out_hbm.at[pl.ds(base + g * _CHUNK, _CHUNK)],
            )
            return carry

        lax.fori_loop(0, n_chunks, body, 0)

    return gather_kernel


def kernel(inputs, table):
    batch, hist = inputs.shape
    total = batch * hist
    idx = inputs.reshape(total).astype(jnp.int32)
    tbl = jnp.pad(table, ((0, 0), (0, _ROW - _EMBED_DIM)))
    out = _make_gather(total)(idx, tbl)
    return out[:, :_EMBED_DIM].reshape(batch, hist, _EMBED_DIM)
